# baseline (speedup 1.0000x reference)
import functools

import jax
import jax.numpy as jnp
from jax import lax
from jax.experimental import pallas as pl
from jax.experimental.pallas import tpu as pltpu
from jax.experimental.pallas import tpu_sc as plsc

_LANES = 128


def _sc_gather(pos_flat, table_flat, n, v, d):
    nw = 32
    per_w = n // nw
    ch = 1024
    n_chunks = per_w // ch
    mesh = plsc.VectorSubcoreMesh(core_axis_name="c", subcore_axis_name="s")

    @functools.partial(
        pl.kernel,
        mesh=mesh,
        compiler_params=pltpu.CompilerParams(needs_layout_passes=False),
        out_type=jax.ShapeDtypeStruct((n * d,), jnp.float32),
        scratch_types=[
            pltpu.VMEM((v * d,), jnp.float32),
            pltpu.VMEM((ch,), jnp.int32),
            pltpu.VMEM((ch * d,), jnp.float32),
        ],
    )
    def gather_kernel(pos_hbm, table_hbm, out_hbm, table_v, idx_v, rows_v):
        wid = lax.axis_index("s") * 2 + lax.axis_index("c")
        base = wid * per_w
        pltpu.sync_copy(table_hbm, table_v)

        def chunk_body(s, carry):
            tok0 = base + s * ch

            pltpu.sync_copy(pos_hbm.at[pl.ds(tok0, ch)], idx_v)
            dst_base = lax.iota(jnp.int32, 16) * d

            def grp_body(g, c2):
                src_base = idx_v[pl.ds(g * 16, 16)] * d
                for e in range(d):
                    val = plsc.load_gather(table_v, [src_base + e])
                    plsc.store_scatter(rows_v, [dst_base + (g * 16 * d + e)], val)
                return c2

            lax.fori_loop(0, ch // 16, grp_body, 0)
            pltpu.sync_copy(rows_v, out_hbm.at[pl.ds(tok0 * d, ch * d)])
            return carry

        lax.fori_loop(0, n_chunks, chunk_body, 0)

    return gather_kernel(pos_flat, table_flat)


def _tc_combine(x_flat, posemb_flat, wt, b2d):
    n, di = x_flat.shape
    de = wt.shape[1]
    tb = 4096

    def body(x_ref, pe_ref, wt_ref, b_ref, o_ref):
        o_ref[...] = (
            jnp.dot(x_ref[...], wt_ref[...], preferred_element_type=jnp.float32)
            + b_ref[...]
            + pe_ref[...]
        )

    return pl.pallas_call(
        body,
        grid=(n // tb,),
        in_specs=[
            pl.BlockSpec((tb, di), lambda i: (i, 0)),
            pl.BlockSpec((tb, de), lambda i: (i, 0)),
            pl.BlockSpec((di, de), lambda i: (0, 0)),
            pl.BlockSpec((1, de), lambda i: (0, 0)),
        ],
        out_specs=pl.BlockSpec((tb, de), lambda i: (i, 0)),
        out_shape=jax.ShapeDtypeStruct((n, de), jnp.float32),
    )(x_flat, posemb_flat, wt, b2d)


def kernel(x, positions, pos_table, W, b):
    bsz, seq, di = x.shape
    de = W.shape[0]
    n = bsz * seq
    x_flat = x.reshape(n, di)
    pos_flat = positions.reshape(n).astype(jnp.int32)
    posemb = _sc_gather(pos_flat, pos_table.reshape(-1), n, pos_table.shape[0], de)
    out = _tc_combine(x_flat, posemb.reshape(n, de), W.T, b.reshape(1, de))
    return out.reshape(bsz, seq, de)


# speedup vs baseline: 1.2249x; 1.2249x over previous
import functools

import jax
import jax.numpy as jnp
from jax import lax
from jax.experimental import pallas as pl
from jax.experimental.pallas import tpu as pltpu
from jax.experimental.pallas import tpu_sc as plsc

_LANES = 128


def _sc_gather(pos_flat, table_flat, n, v, d):
    nw = 32
    per_w = n // nw
    ch = 1024
    n_chunks = per_w // ch
    mesh = plsc.VectorSubcoreMesh(core_axis_name="c", subcore_axis_name="s")

    @functools.partial(
        pl.kernel,
        mesh=mesh,
        compiler_params=pltpu.CompilerParams(needs_layout_passes=False),
        out_type=jax.ShapeDtypeStruct((n * d,), jnp.float32),
        scratch_types=[
            pltpu.VMEM((v * d,), jnp.float32),
            pltpu.VMEM((ch,), jnp.int32),
            pltpu.VMEM((ch * d,), jnp.float32),
        ],
    )
    def gather_kernel(pos_hbm, table_hbm, out_hbm, table_v, idx_v, rows_v):
        wid = lax.axis_index("s") * 2 + lax.axis_index("c")
        base = wid * per_w
        pltpu.sync_copy(table_hbm, table_v)

        def chunk_body(s, carry):
            tok0 = base + s * ch

            pltpu.sync_copy(pos_hbm.at[pl.ds(tok0, ch)], idx_v)
            dst_base = lax.iota(jnp.int32, 16) * d

            @plsc.parallel_loop(0, ch // 16, unroll=4)
            def grp_body(g):
                src_base = idx_v[pl.ds(g * 16, 16)] * d
                for e in range(d):
                    val = plsc.load_gather(table_v, [src_base + e])
                    plsc.store_scatter(rows_v, [dst_base + (g * 16 * d + e)], val)
            pltpu.sync_copy(rows_v, out_hbm.at[pl.ds(tok0 * d, ch * d)])
            return carry

        lax.fori_loop(0, n_chunks, chunk_body, 0)

    return gather_kernel(pos_flat, table_flat)


def _tc_combine(x_flat, posemb_flat, wt, b2d):
    n, di = x_flat.shape
    de = wt.shape[1]
    tb = 4096

    def body(x_ref, pe_ref, wt_ref, b_ref, o_ref):
        o_ref[...] = (
            jnp.dot(x_ref[...], wt_ref[...], preferred_element_type=jnp.float32)
            + b_ref[...]
            + pe_ref[...]
        )

    return pl.pallas_call(
        body,
        grid=(n // tb,),
        in_specs=[
            pl.BlockSpec((tb, di), lambda i: (i, 0)),
            pl.BlockSpec((tb, de), lambda i: (i, 0)),
            pl.BlockSpec((di, de), lambda i: (0, 0)),
            pl.BlockSpec((1, de), lambda i: (0, 0)),
        ],
        out_specs=pl.BlockSpec((tb, de), lambda i: (i, 0)),
        out_shape=jax.ShapeDtypeStruct((n, de), jnp.float32),
    )(x_flat, posemb_flat, wt, b2d)


def kernel(x, positions, pos_table, W, b):
    bsz, seq, di = x.shape
    de = W.shape[0]
    n = bsz * seq
    x_flat = x.reshape(n, di)
    pos_flat = positions.reshape(n).astype(jnp.int32)
    posemb = _sc_gather(pos_flat, pos_table.reshape(-1), n, pos_table.shape[0], de)
    out = _tc_combine(x_flat, posemb.reshape(n, de), W.T, b.reshape(1, de))
    return out.reshape(bsz, seq, de)


# speedup vs baseline: 1.9428x; 1.5861x over previous
import functools

import jax
import jax.numpy as jnp
from jax import lax
from jax.experimental import pallas as pl
from jax.experimental.pallas import tpu as pltpu
from jax.experimental.pallas import tpu_sc as plsc

_LANES = 128


def _sc_gather(pos_flat, table_flat, n, v, d):
    nw = 32
    per_w = n // nw
    ch = 1024
    n_chunks = per_w // ch
    mesh = plsc.VectorSubcoreMesh(core_axis_name="c", subcore_axis_name="s")

    @functools.partial(
        pl.kernel,
        mesh=mesh,
        compiler_params=pltpu.CompilerParams(needs_layout_passes=False),
        out_type=jax.ShapeDtypeStruct((n * d,), jnp.float32),
        scratch_types=[
            pltpu.VMEM((v * d,), jnp.float32),
            pltpu.VMEM((ch,), jnp.int32),
            pltpu.VMEM((ch * d,), jnp.float32),
        ],
    )
    def gather_kernel(pos_hbm, table_hbm, out_hbm, table_v, idx_v, rows_v):
        wid = lax.axis_index("s") * 2 + lax.axis_index("c")
        base = wid * per_w
        pltpu.sync_copy(table_hbm, table_v)

        def chunk_body(s, carry):
            tok0 = base + s * ch

            pltpu.sync_copy(pos_hbm.at[pl.ds(tok0, ch)], idx_v)
            lane = lax.iota(jnp.int32, 16)

            @plsc.parallel_loop(0, ch // 16, unroll=2)
            def grp_body(g):
                for j in range(16):
                    t = g * 16 + j
                    pj = plsc.load_gather(idx_v, [jnp.full((16,), t, jnp.int32)])
                    src = pj * d + lane
                    rows_v[pl.ds(t * d, 16)] = plsc.load_gather(table_v, [src])
                    rows_v[pl.ds(t * d + 16, 16)] = plsc.load_gather(
                        table_v, [src + 16]
                    )
            pltpu.sync_copy(rows_v, out_hbm.at[pl.ds(tok0 * d, ch * d)])
            return carry

        lax.fori_loop(0, n_chunks, chunk_body, 0)

    return gather_kernel(pos_flat, table_flat)


def _tc_combine(x_flat, posemb_flat, wt, b2d):
    n, di = x_flat.shape
    de = wt.shape[1]
    tb = 4096

    def body(x_ref, pe_ref, wt_ref, b_ref, o_ref):
        o_ref[...] = (
            jnp.dot(x_ref[...], wt_ref[...], preferred_element_type=jnp.float32)
            + b_ref[...]
            + pe_ref[...]
        )

    return pl.pallas_call(
        body,
        grid=(n // tb,),
        in_specs=[
            pl.BlockSpec((tb, di), lambda i: (i, 0)),
            pl.BlockSpec((tb, de), lambda i: (i, 0)),
            pl.BlockSpec((di, de), lambda i: (0, 0)),
            pl.BlockSpec((1, de), lambda i: (0, 0)),
        ],
        out_specs=pl.BlockSpec((tb, de), lambda i: (i, 0)),
        out_shape=jax.ShapeDtypeStruct((n, de), jnp.float32),
    )(x_flat, posemb_flat, wt, b2d)


def kernel(x, positions, pos_table, W, b):
    bsz, seq, di = x.shape
    de = W.shape[0]
    n = bsz * seq
    x_flat = x.reshape(n, di)
    pos_flat = positions.reshape(n).astype(jnp.int32)
    posemb = _sc_gather(pos_flat, pos_table.reshape(-1), n, pos_table.shape[0], de)
    out = _tc_combine(x_flat, posemb.reshape(n, de), W.T, b.reshape(1, de))
    return out.reshape(bsz, seq, de)
